# baseline (speedup 1.0000x reference)
import functools
import jax
import jax.numpy as jnp
from jax import lax
from jax.experimental import pallas as pl
from jax.experimental.pallas import tpu as pltpu, tpu_sc as plsc

BATCH = 16384
HIST = 50
HIST_PAD = 64
EMBED = 32
DICT = 1000000

NC = 2
NS = 16
NW = NC * NS
LANES = 16

B_PER_W = BATCH // NW
CB = 64
NCHUNK = B_PER_W // CB

_mesh = plsc.VectorSubcoreMesh(core_axis_name="c", subcore_axis_name="s")


@functools.partial(
    pl.kernel,
    out_type=jax.ShapeDtypeStruct((BATCH, EMBED), jnp.float32),
    mesh=_mesh,
    compiler_params=pltpu.CompilerParams(use_tc_tiling_on_sc=False,
                                         needs_layout_passes=False),
    scratch_types=[
        pltpu.VMEM((CB, HIST), jnp.int32),
        pltpu.VMEM((HIST * CB,), jnp.int32),
        pltpu.VMEM((HIST, CB, EMBED), jnp.float32),
        pltpu.VMEM((CB, EMBED), jnp.float32),
        pltpu.SemaphoreType.DMA,
    ],
)
def _user_encoder(x_hbm, table_hbm, out_hbm, idx_raw, idx_t, rows_v, out_v,
                  sem):
  wid = lax.axis_index("s") * NC + lax.axis_index("c")

  lane_cb = lax.iota(jnp.int32, LANES) * CB

  def chunk_body(c, _):
    b0 = c * CB

    pltpu.sync_copy(x_hbm.at[pl.ds(wid * B_PER_W + b0, CB)], idx_raw)

    def transp(r, _):
      for o in (0, 16, 32, HIST - LANES):
        v = idx_raw[r, pl.ds(o, LANES)]
        plsc.store_scatter(idx_t, [lane_cb + (o * CB + r)], v)
      return 0
    lax.fori_loop(0, CB, transp, 0)

    def fire(l, _):
      pltpu.async_copy(table_hbm.at[idx_t.at[pl.ds(l * CB, CB)]],
                       rows_v.at[l], sem)
      return 0
    lax.fori_loop(0, HIST, fire, 0)

    def drain(l, _):
      pltpu.make_async_copy(table_hbm.at[idx_t.at[pl.ds(l * CB, CB)]],
                            rows_v.at[l], sem).wait()
      return 0
    lax.fori_loop(0, HIST, drain, 0)

    def pool(i, _):
      acc0 = rows_v[0, i, 0:16]
      acc1 = rows_v[0, i, 16:32]
      for l in range(1, HIST):
        acc0 = acc0 + rows_v[l, i, 0:16]
        acc1 = acc1 + rows_v[l, i, 16:32]
      scale = jnp.float32(1.0 / HIST)
      out_v[i, 0:16] = acc0 * scale
      out_v[i, 16:32] = acc1 * scale
      return 0
    lax.fori_loop(0, CB, pool, 0)

    pltpu.sync_copy(out_v, out_hbm.at[pl.ds(wid * B_PER_W + b0, CB)])
    return 0

  lax.fori_loop(0, NCHUNK, chunk_body, 0)


def kernel(x, table):
  return _user_encoder(x.astype(jnp.int32), table)


# speedup vs baseline: 1.0139x; 1.0139x over previous
import functools
import jax
import jax.numpy as jnp
from jax import lax
from jax.experimental import pallas as pl
from jax.experimental.pallas import tpu as pltpu, tpu_sc as plsc

BATCH = 16384
HIST = 50
HIST_PAD = 64
EMBED = 32
DICT = 1000000

NC = 2
NS = 16
NW = NC * NS
LANES = 16

B_PER_W = BATCH // NW
CB = 64
NCHUNK = B_PER_W // CB

_mesh = plsc.VectorSubcoreMesh(core_axis_name="c", subcore_axis_name="s")


@functools.partial(
    pl.kernel,
    out_type=jax.ShapeDtypeStruct((BATCH, EMBED), jnp.float32),
    mesh=_mesh,
    compiler_params=pltpu.CompilerParams(use_tc_tiling_on_sc=False,
                                         needs_layout_passes=False),
    scratch_types=[
        pltpu.VMEM((CB * HIST // 32, 32), jnp.int32),
        pltpu.VMEM((HIST * CB,), jnp.int32),
        pltpu.VMEM((HIST, CB, EMBED), jnp.float32),
        pltpu.VMEM((CB, EMBED), jnp.float32),
        pltpu.SemaphoreType.DMA,
    ],
)
def _user_encoder(x_hbm, table_hbm, out_hbm, idx_raw, idx_t, rows_v, out_v,
                  sem):
  wid = lax.axis_index("s") * NC + lax.axis_index("c")

  lane = lax.iota(jnp.int32, LANES)
  raw_rows = CB * HIST // 32

  def chunk_body(c, _):
    b0 = c * CB

    pltpu.sync_copy(
        x_hbm.at[pl.ds((wid * B_PER_W + b0) * HIST // 32, raw_rows)],
        idx_raw)

    def transp(rr, _):
      for o in (0, 16):
        v = idx_raw[rr, pl.ds(o, LANES)]
        p = rr * 32 + o + lane
        r = p // HIST
        l = p - r * HIST
        plsc.store_scatter(idx_t, [l * CB + r], v)
      return 0
    lax.fori_loop(0, raw_rows, transp, 0)

    def fire(l, _):
      pltpu.async_copy(table_hbm.at[idx_t.at[pl.ds(l * CB, CB)]],
                       rows_v.at[l], sem)
      return 0
    lax.fori_loop(0, HIST, fire, 0)

    def drain(l, _):
      pltpu.make_async_copy(table_hbm.at[idx_t.at[pl.ds(l * CB, CB)]],
                            rows_v.at[l], sem).wait()
      return 0
    lax.fori_loop(0, HIST, drain, 0)

    def pool(i, _):
      acc0 = rows_v[0, i, 0:16]
      acc1 = rows_v[0, i, 16:32]
      for l in range(1, HIST):
        acc0 = acc0 + rows_v[l, i, 0:16]
        acc1 = acc1 + rows_v[l, i, 16:32]
      scale = jnp.float32(1.0 / HIST)
      out_v[i, 0:16] = acc0 * scale
      out_v[i, 16:32] = acc1 * scale
      return 0
    lax.fori_loop(0, CB, pool, 0)

    pltpu.sync_copy(out_v, out_hbm.at[pl.ds(wid * B_PER_W + b0, CB)])
    return 0

  lax.fori_loop(0, NCHUNK, chunk_body, 0)


def kernel(x, table):
  x2 = x.astype(jnp.int32).reshape(BATCH * HIST // 32, 32)
  return _user_encoder(x2, table)
